# baseline (speedup 1.0000x reference)
import functools

import jax
import jax.numpy as jnp
from jax import lax
from jax.experimental import pallas as pl
from jax.experimental.pallas import tpu as pltpu
from jax.experimental.pallas import tpu_sc as plsc

_N_LEVELS = 16
_T = 1 << 15
_MASK = _T - 1
_HASH_PRIME = -1640531535
_NC = 2
_LANES = 16
_CHUNK = 8192
_VU = 4

_P1, _P2, _P4, _P8 = 1.5, 2.25, 5.0625, 25.62890625


def _encode_body(part_base, npts,
                 ux_hbm, uy_hbm, tab_hbm, out_hbm,
                 tab_v, ux0_v, uy0_v, e0_v, ux1_v, uy1_v, e1_v,
                 sin0, sin1, sout0, sout1):
    half_n = npts // _NC
    nchunks = half_n // _CHUNK
    c = lax.axis_index("c")
    s = lax.axis_index("s")
    lvl = s
    pltpu.sync_copy(tab_hbm.at[lvl], tab_v)

    bufs = ((ux0_v, uy0_v, e0_v, sin0, sout0),
            (ux1_v, uy1_v, e1_v, sin1, sout1))

    lv = jnp.full((_LANES,), lvl, dtype=jnp.int32)
    scale = jnp.full((_LANES,), 16.0, dtype=jnp.float32)
    scale = scale * jnp.where((lv & 1) != 0, _P1, 1.0).astype(jnp.float32)
    scale = scale * jnp.where((lv & 2) != 0, _P2, 1.0).astype(jnp.float32)
    scale = scale * jnp.where((lv & 4) != 0, _P4, 1.0).astype(jnp.float32)
    scale = scale * jnp.where((lv & 8) != 0, _P8, 1.0).astype(jnp.float32)

    in_base0 = part_base + c * half_n
    out_base0 = c * half_n

    def in_copies(p, ci):
        ux_v, uy_v = bufs[p][0], bufs[p][1]
        sem = bufs[p][3]
        base = in_base0 + ci * _CHUNK
        return (pltpu.make_async_copy(ux_hbm.at[pl.ds(base, _CHUNK)], ux_v, sem),
                pltpu.make_async_copy(uy_hbm.at[pl.ds(base, _CHUNK)], uy_v, sem))

    def out_copies(p, ci):
        e_v = bufs[p][2]
        sem = bufs[p][4]
        base = out_base0 + ci * _CHUNK
        return (pltpu.make_async_copy(e_v, out_hbm.at[lvl, pl.ds(base, _CHUNK)], sem),)

    for cp in in_copies(0, 0) + in_copies(1, 1):
        cp.start()

    def encode_16(x, y):
        px = x * scale
        py = y * scale
        xi = px.astype(jnp.int32)
        yi = py.astype(jnp.int32)
        wx = px - xi.astype(jnp.float32)
        wy = py - yi.astype(jnp.float32)
        hy0 = yi * _HASH_PRIME
        hy1 = hy0 + _HASH_PRIME
        x1 = xi + 1
        i00 = (xi ^ hy0) & _MASK
        i01 = (xi ^ hy1) & _MASK
        i10 = (x1 ^ hy0) & _MASK
        i11 = (x1 ^ hy1) & _MASK
        wx0 = 1.0 - wx
        wy0 = 1.0 - wy
        w00 = wx0 * wy0
        w01 = wx0 * wy
        w10 = wx * wy0
        w11 = wx * wy
        g00 = plsc.load_gather(tab_v, [i00])
        g01 = plsc.load_gather(tab_v, [i01])
        g10 = plsc.load_gather(tab_v, [i10])
        g11 = plsc.load_gather(tab_v, [i11])

        def f0(g):
            return plsc.bitcast(g << 16, jnp.float32)

        def f1(g):
            return plsc.bitcast(g & jnp.int32(-65536), jnp.float32)

        e0 = f0(g00) * w00 + f0(g01) * w01 + f0(g10) * w10 + f0(g11) * w11
        e1 = f1(g00) * w00 + f1(g01) * w01 + f1(g10) * w10 + f1(g11) * w11
        return e0, e1

    def chunk_pair(ci2, carry):
        for p in (0, 1):
            ci = ci2 * 2 + p
            ux_v, uy_v, e_v = bufs[p][0], bufs[p][1], bufs[p][2]
            for cp in in_copies(p, ci):
                cp.wait()

            @pl.when(ci2 > 0)
            def _wait_prev_out():
                for cp in out_copies(p, ci):
                    cp.wait()

            @plsc.parallel_loop(0, _CHUNK // _LANES, unroll=_VU)
            def vec_body(i):
                off = i * _LANES
                x = ux_v[pl.ds(off, _LANES)]
                y = uy_v[pl.ds(off, _LANES)]
                e0, e1 = encode_16(x, y)
                pk = plsc.pack(e0, e1, format=plsc.PackFormat.INTERLEAVED)
                e_v[pl.ds(off, _LANES)] = plsc.bitcast(pk, jnp.int32)

            for cp in out_copies(p, ci):
                cp.start()

            @pl.when(ci + 2 < nchunks)
            def _prefetch_next():
                for cp in in_copies(p, ci + 2):
                    cp.start()

        return carry

    lax.fori_loop(0, nchunks // 2, chunk_pair, 0)
    for p in (0, 1):
        for cp in out_copies(p, nchunks - 2 + p):
            cp.wait()


@functools.lru_cache(maxsize=None)
def _make_encode(n, nsplit, part):
    npts = n // nsplit
    mesh = plsc.VectorSubcoreMesh(core_axis_name="c", subcore_axis_name="s")
    return functools.partial(
        pl.kernel,
        mesh=mesh,
        out_type=jax.ShapeDtypeStruct((_N_LEVELS, npts), jnp.int32),
        scratch_types=[
            pltpu.VMEM((_T,), jnp.int32),
            pltpu.VMEM((_CHUNK,), jnp.float32),
            pltpu.VMEM((_CHUNK,), jnp.float32),
            pltpu.VMEM((_CHUNK,), jnp.int32),
            pltpu.VMEM((_CHUNK,), jnp.float32),
            pltpu.VMEM((_CHUNK,), jnp.float32),
            pltpu.VMEM((_CHUNK,), jnp.int32),
            pltpu.SemaphoreType.DMA,
            pltpu.SemaphoreType.DMA,
            pltpu.SemaphoreType.DMA,
            pltpu.SemaphoreType.DMA,
        ],
        compiler_params=pltpu.CompilerParams(needs_layout_passes=False),
    )(functools.partial(_encode_body, part * npts, npts))


def _mlp_body(x_ref, w1ta_ref, w1tb_ref, w2t_ref, w3t_ref, o_ref):
    x = x_ref[...]
    x0 = lax.bitcast_convert_type(x << 16, jnp.float32)
    x1 = lax.bitcast_convert_type(x & jnp.int32(-65536), jnp.float32)
    h = (lax.dot_general(w1ta_ref[...], x0, (((1,), (0,)), ((), ())),
                         preferred_element_type=jnp.float32)
         + lax.dot_general(w1tb_ref[...], x1, (((1,), (0,)), ((), ())),
                           preferred_element_type=jnp.float32))
    h = jnp.maximum(h, 0.0)
    h = lax.dot_general(w2t_ref[...], h, (((1,), (0,)), ((), ())),
                        preferred_element_type=jnp.float32)
    h = jnp.maximum(h, 0.0)
    o_ref[...] = lax.dot_general(w3t_ref[...], h, (((1,), (0,)), ((), ())),
                                 preferred_element_type=jnp.float32)


_BN = 16384


def _mlp(enc, w1ta, w1tb, w2t, w3t):
    npts = enc.shape[1]
    bn = min(_BN, npts)
    return pl.pallas_call(
        _mlp_body,
        grid=(npts // bn,),
        in_specs=[
            pl.BlockSpec((_N_LEVELS, bn), lambda i: (0, i)),
            pl.BlockSpec((64, _N_LEVELS), lambda i: (0, 0)),
            pl.BlockSpec((64, _N_LEVELS), lambda i: (0, 0)),
            pl.BlockSpec((64, 64), lambda i: (0, 0)),
            pl.BlockSpec((3, 64), lambda i: (0, 0)),
        ],
        out_specs=pl.BlockSpec((3, bn), lambda i: (0, i)),
        out_shape=jax.ShapeDtypeStruct((3, npts), jnp.float32),
    )(enc, w1ta, w1tb, w2t, w3t)


_NSPLIT = 2


def kernel(uv_coords, bake, table, W1, W2, W3):
    del bake
    n = uv_coords.shape[0]
    ux = uv_coords[:, 0]
    uy = uv_coords[:, 1]
    tabp = lax.bitcast_convert_type(table.astype(jnp.bfloat16), jnp.int32)
    w1t = W1.T
    w1ta, w1tb = w1t[:, 0::2], w1t[:, 1::2]
    w2t, w3t = W2.T, W3.T
    outs = []
    for part in range(_NSPLIT):
        enc = _make_encode(n, _NSPLIT, part)(ux, uy, tabp)
        outs.append(_mlp(enc, w1ta, w1tb, w2t, w3t).T)
    out = jnp.concatenate(outs, axis=0) if len(outs) > 1 else outs[0]
    return out.astype(jnp.float32)


# speedup vs baseline: 1.0213x; 1.0213x over previous
import functools

import jax
import jax.numpy as jnp
from jax import lax
from jax.experimental import pallas as pl
from jax.experimental.pallas import tpu as pltpu
from jax.experimental.pallas import tpu_sc as plsc

_N_LEVELS = 16
_T = 1 << 15
_MASK = _T - 1
_HASH_PRIME = -1640531535
_NC = 2
_LANES = 16
_CHUNK = 8192
_VU = 6

_P1, _P2, _P4, _P8 = 1.5, 2.25, 5.0625, 25.62890625


def _encode_body(part_base, npts,
                 ux_hbm, uy_hbm, tab_hbm, out_hbm,
                 tab_v, ux0_v, uy0_v, e0_v, ux1_v, uy1_v, e1_v,
                 sin0, sin1, sout0, sout1):
    half_n = npts // _NC
    nchunks = half_n // _CHUNK
    c = lax.axis_index("c")
    s = lax.axis_index("s")
    lvl = s
    pltpu.sync_copy(tab_hbm.at[lvl], tab_v)

    bufs = ((ux0_v, uy0_v, e0_v, sin0, sout0),
            (ux1_v, uy1_v, e1_v, sin1, sout1))

    lv = jnp.full((_LANES,), lvl, dtype=jnp.int32)
    scale = jnp.full((_LANES,), 16.0, dtype=jnp.float32)
    scale = scale * jnp.where((lv & 1) != 0, _P1, 1.0).astype(jnp.float32)
    scale = scale * jnp.where((lv & 2) != 0, _P2, 1.0).astype(jnp.float32)
    scale = scale * jnp.where((lv & 4) != 0, _P4, 1.0).astype(jnp.float32)
    scale = scale * jnp.where((lv & 8) != 0, _P8, 1.0).astype(jnp.float32)

    in_base0 = part_base + c * half_n
    out_base0 = c * half_n

    def in_copies(p, ci):
        ux_v, uy_v = bufs[p][0], bufs[p][1]
        sem = bufs[p][3]
        base = in_base0 + ci * _CHUNK
        return (pltpu.make_async_copy(ux_hbm.at[pl.ds(base, _CHUNK)], ux_v, sem),
                pltpu.make_async_copy(uy_hbm.at[pl.ds(base, _CHUNK)], uy_v, sem))

    def out_copies(p, ci):
        e_v = bufs[p][2]
        sem = bufs[p][4]
        base = out_base0 + ci * _CHUNK
        return (pltpu.make_async_copy(e_v, out_hbm.at[lvl, pl.ds(base, _CHUNK)], sem),)

    for cp in in_copies(0, 0) + in_copies(1, 1):
        cp.start()

    def encode_16(x, y):
        px = x * scale
        py = y * scale
        xi = px.astype(jnp.int32)
        yi = py.astype(jnp.int32)
        wx = px - xi.astype(jnp.float32)
        wy = py - yi.astype(jnp.float32)
        hy0 = yi * _HASH_PRIME
        hy1 = hy0 + _HASH_PRIME
        x1 = xi + 1
        i00 = (xi ^ hy0) & _MASK
        i01 = (xi ^ hy1) & _MASK
        i10 = (x1 ^ hy0) & _MASK
        i11 = (x1 ^ hy1) & _MASK
        wx0 = 1.0 - wx
        wy0 = 1.0 - wy
        w00 = wx0 * wy0
        w01 = wx0 * wy
        w10 = wx * wy0
        w11 = wx * wy
        g00 = plsc.load_gather(tab_v, [i00])
        g01 = plsc.load_gather(tab_v, [i01])
        g10 = plsc.load_gather(tab_v, [i10])
        g11 = plsc.load_gather(tab_v, [i11])

        def f0(g):
            return plsc.bitcast(g << 16, jnp.float32)

        def f1(g):
            return plsc.bitcast(g & jnp.int32(-65536), jnp.float32)

        e0 = f0(g00) * w00 + f0(g01) * w01 + f0(g10) * w10 + f0(g11) * w11
        e1 = f1(g00) * w00 + f1(g01) * w01 + f1(g10) * w10 + f1(g11) * w11
        return e0, e1

    def chunk_pair(ci2, carry):
        for p in (0, 1):
            ci = ci2 * 2 + p
            ux_v, uy_v, e_v = bufs[p][0], bufs[p][1], bufs[p][2]
            for cp in in_copies(p, ci):
                cp.wait()

            @pl.when(ci2 > 0)
            def _wait_prev_out():
                for cp in out_copies(p, ci):
                    cp.wait()

            @plsc.parallel_loop(0, _CHUNK // _LANES, unroll=_VU)
            def vec_body(i):
                off = i * _LANES
                x = ux_v[pl.ds(off, _LANES)]
                y = uy_v[pl.ds(off, _LANES)]
                e0, e1 = encode_16(x, y)
                pk = plsc.pack(e0, e1, format=plsc.PackFormat.INTERLEAVED)
                e_v[pl.ds(off, _LANES)] = plsc.bitcast(pk, jnp.int32)

            for cp in out_copies(p, ci):
                cp.start()

            @pl.when(ci + 2 < nchunks)
            def _prefetch_next():
                for cp in in_copies(p, ci + 2):
                    cp.start()

        return carry

    lax.fori_loop(0, nchunks // 2, chunk_pair, 0)
    for p in (0, 1):
        for cp in out_copies(p, nchunks - 2 + p):
            cp.wait()


@functools.lru_cache(maxsize=None)
def _make_encode(n, nsplit, part):
    npts = n // nsplit
    mesh = plsc.VectorSubcoreMesh(core_axis_name="c", subcore_axis_name="s")
    return functools.partial(
        pl.kernel,
        mesh=mesh,
        out_type=jax.ShapeDtypeStruct((_N_LEVELS, npts), jnp.int32),
        scratch_types=[
            pltpu.VMEM((_T,), jnp.int32),
            pltpu.VMEM((_CHUNK,), jnp.float32),
            pltpu.VMEM((_CHUNK,), jnp.float32),
            pltpu.VMEM((_CHUNK,), jnp.int32),
            pltpu.VMEM((_CHUNK,), jnp.float32),
            pltpu.VMEM((_CHUNK,), jnp.float32),
            pltpu.VMEM((_CHUNK,), jnp.int32),
            pltpu.SemaphoreType.DMA,
            pltpu.SemaphoreType.DMA,
            pltpu.SemaphoreType.DMA,
            pltpu.SemaphoreType.DMA,
        ],
        compiler_params=pltpu.CompilerParams(needs_layout_passes=False),
    )(functools.partial(_encode_body, part * npts, npts))


def _mlp_body(x_ref, w1ta_ref, w1tb_ref, w2t_ref, w3t_ref, o_ref):
    x = x_ref[...]
    x0 = lax.bitcast_convert_type(x << 16, jnp.float32)
    x1 = lax.bitcast_convert_type(x & jnp.int32(-65536), jnp.float32)
    h = (lax.dot_general(w1ta_ref[...], x0, (((1,), (0,)), ((), ())),
                         preferred_element_type=jnp.float32)
         + lax.dot_general(w1tb_ref[...], x1, (((1,), (0,)), ((), ())),
                           preferred_element_type=jnp.float32))
    h = jnp.maximum(h, 0.0)
    h = lax.dot_general(w2t_ref[...], h, (((1,), (0,)), ((), ())),
                        preferred_element_type=jnp.float32)
    h = jnp.maximum(h, 0.0)
    o_ref[...] = lax.dot_general(w3t_ref[...], h, (((1,), (0,)), ((), ())),
                                 preferred_element_type=jnp.float32)


_BN = 16384


def _mlp(enc, w1ta, w1tb, w2t, w3t):
    npts = enc.shape[1]
    bn = min(_BN, npts)
    return pl.pallas_call(
        _mlp_body,
        grid=(npts // bn,),
        in_specs=[
            pl.BlockSpec((_N_LEVELS, bn), lambda i: (0, i)),
            pl.BlockSpec((64, _N_LEVELS), lambda i: (0, 0)),
            pl.BlockSpec((64, _N_LEVELS), lambda i: (0, 0)),
            pl.BlockSpec((64, 64), lambda i: (0, 0)),
            pl.BlockSpec((3, 64), lambda i: (0, 0)),
        ],
        out_specs=pl.BlockSpec((3, bn), lambda i: (0, i)),
        out_shape=jax.ShapeDtypeStruct((3, npts), jnp.float32),
    )(enc, w1ta, w1tb, w2t, w3t)


_NSPLIT = 1


def kernel(uv_coords, bake, table, W1, W2, W3):
    del bake
    n = uv_coords.shape[0]
    ux = uv_coords[:, 0]
    uy = uv_coords[:, 1]
    tabp = lax.bitcast_convert_type(table.astype(jnp.bfloat16), jnp.int32)
    w1t = W1.T
    w1ta, w1tb = w1t[:, 0::2], w1t[:, 1::2]
    w2t, w3t = W2.T, W3.T
    outs = []
    for part in range(_NSPLIT):
        enc = _make_encode(n, _NSPLIT, part)(ux, uy, tabp)
        outs.append(_mlp(enc, w1ta, w1tb, w2t, w3t).T)
    out = jnp.concatenate(outs, axis=0) if len(outs) > 1 else outs[0]
    return out.astype(jnp.float32)


# speedup vs baseline: 1.0251x; 1.0038x over previous
import functools

import jax
import jax.numpy as jnp
from jax import lax
from jax.experimental import pallas as pl
from jax.experimental.pallas import tpu as pltpu
from jax.experimental.pallas import tpu_sc as plsc

_N_LEVELS = 16
_T = 1 << 15
_MASK = _T - 1
_HASH_PRIME = -1640531535
_NC = 2
_LANES = 16
_CHUNK = 8192
_VU = 6

_P1, _P2, _P4, _P8 = 1.5, 2.25, 5.0625, 25.62890625


def _encode_body(part_base, npts,
                 ux_hbm, uy_hbm, tab_hbm, out_hbm,
                 tab_v, ux0_v, uy0_v, e0_v, ux1_v, uy1_v, e1_v,
                 sin0, sin1, sout0, sout1):
    half_n = npts // _NC
    nchunks = half_n // _CHUNK
    c = lax.axis_index("c")
    s = lax.axis_index("s")
    lvl = s
    pltpu.sync_copy(tab_hbm.at[lvl], tab_v)

    bufs = ((ux0_v, uy0_v, e0_v, sin0, sout0),
            (ux1_v, uy1_v, e1_v, sin1, sout1))

    lv = jnp.full((_LANES,), lvl, dtype=jnp.int32)
    scale = jnp.full((_LANES,), 16.0, dtype=jnp.float32)
    scale = scale * jnp.where((lv & 1) != 0, _P1, 1.0).astype(jnp.float32)
    scale = scale * jnp.where((lv & 2) != 0, _P2, 1.0).astype(jnp.float32)
    scale = scale * jnp.where((lv & 4) != 0, _P4, 1.0).astype(jnp.float32)
    scale = scale * jnp.where((lv & 8) != 0, _P8, 1.0).astype(jnp.float32)

    in_base0 = part_base + c * half_n
    out_base0 = c * half_n

    def in_copies(p, ci):
        ux_v, uy_v = bufs[p][0], bufs[p][1]
        sem = bufs[p][3]
        base = in_base0 + ci * _CHUNK
        return (pltpu.make_async_copy(ux_hbm.at[pl.ds(base, _CHUNK)], ux_v, sem),
                pltpu.make_async_copy(uy_hbm.at[pl.ds(base, _CHUNK)], uy_v, sem))

    def out_copies(p, ci):
        e_v = bufs[p][2]
        sem = bufs[p][4]
        base = out_base0 + ci * _CHUNK
        return (pltpu.make_async_copy(e_v, out_hbm.at[lvl, pl.ds(base, _CHUNK)], sem),)

    for cp in in_copies(0, 0) + in_copies(1, 1):
        cp.start()

    def encode_16(x, y):
        px = x * scale
        py = y * scale
        xi = px.astype(jnp.int32)
        yi = py.astype(jnp.int32)
        wx = px - xi.astype(jnp.float32)
        wy = py - yi.astype(jnp.float32)
        hy0 = yi * _HASH_PRIME
        hy1 = hy0 + _HASH_PRIME
        x1 = xi + 1
        i00 = (xi ^ hy0) & _MASK
        i01 = (xi ^ hy1) & _MASK
        i10 = (x1 ^ hy0) & _MASK
        i11 = (x1 ^ hy1) & _MASK
        wx0 = 1.0 - wx
        wy0 = 1.0 - wy
        w00 = wx0 * wy0
        w01 = wx0 * wy
        w10 = wx * wy0
        w11 = wx * wy
        g00 = plsc.load_gather(tab_v, [i00])
        g01 = plsc.load_gather(tab_v, [i01])
        g10 = plsc.load_gather(tab_v, [i10])
        g11 = plsc.load_gather(tab_v, [i11])

        def f0(g):
            return plsc.bitcast(g << 16, jnp.float32)

        def f1(g):
            return plsc.bitcast(g & jnp.int32(-65536), jnp.float32)

        e0 = f0(g00) * w00 + f0(g01) * w01 + f0(g10) * w10 + f0(g11) * w11
        e1 = f1(g00) * w00 + f1(g01) * w01 + f1(g10) * w10 + f1(g11) * w11
        return e0, e1

    def chunk_pair(ci2, carry):
        for p in (0, 1):
            ci = ci2 * 2 + p
            ux_v, uy_v, e_v = bufs[p][0], bufs[p][1], bufs[p][2]
            for cp in in_copies(p, ci):
                cp.wait()

            @pl.when(ci2 > 0)
            def _wait_prev_out():
                for cp in out_copies(p, ci):
                    cp.wait()

            @plsc.parallel_loop(0, _CHUNK // _LANES, unroll=_VU)
            def vec_body(i):
                off = i * _LANES
                x = ux_v[pl.ds(off, _LANES)]
                y = uy_v[pl.ds(off, _LANES)]
                e0, e1 = encode_16(x, y)
                pk = plsc.pack(e0, e1, format=plsc.PackFormat.INTERLEAVED)
                e_v[pl.ds(off, _LANES)] = plsc.bitcast(pk, jnp.int32)

            for cp in out_copies(p, ci):
                cp.start()

            @pl.when(ci + 2 < nchunks)
            def _prefetch_next():
                for cp in in_copies(p, ci + 2):
                    cp.start()

        return carry

    lax.fori_loop(0, nchunks // 2, chunk_pair, 0)
    for p in (0, 1):
        for cp in out_copies(p, nchunks - 2 + p):
            cp.wait()


@functools.lru_cache(maxsize=None)
def _make_encode(n, nsplit, part):
    npts = n // nsplit
    mesh = plsc.VectorSubcoreMesh(core_axis_name="c", subcore_axis_name="s")
    return functools.partial(
        pl.kernel,
        mesh=mesh,
        out_type=jax.ShapeDtypeStruct((_N_LEVELS, npts), jnp.int32),
        scratch_types=[
            pltpu.VMEM((_T,), jnp.int32),
            pltpu.VMEM((_CHUNK,), jnp.float32),
            pltpu.VMEM((_CHUNK,), jnp.float32),
            pltpu.VMEM((_CHUNK,), jnp.int32),
            pltpu.VMEM((_CHUNK,), jnp.float32),
            pltpu.VMEM((_CHUNK,), jnp.float32),
            pltpu.VMEM((_CHUNK,), jnp.int32),
            pltpu.SemaphoreType.DMA,
            pltpu.SemaphoreType.DMA,
            pltpu.SemaphoreType.DMA,
            pltpu.SemaphoreType.DMA,
        ],
        compiler_params=pltpu.CompilerParams(needs_layout_passes=False),
    )(functools.partial(_encode_body, part * npts, npts))


def _mlp_body(x_ref, w1ta_ref, w1tb_ref, w2t_ref, w3t_ref, o_ref):
    x = x_ref[...]
    x0 = lax.bitcast_convert_type(x << 16, jnp.float32)
    x1 = lax.bitcast_convert_type(x & jnp.int32(-65536), jnp.float32)
    h = (lax.dot_general(w1ta_ref[...], x0, (((1,), (0,)), ((), ())),
                         preferred_element_type=jnp.float32)
         + lax.dot_general(w1tb_ref[...], x1, (((1,), (0,)), ((), ())),
                           preferred_element_type=jnp.float32))
    h = jnp.maximum(h, 0.0)
    h = lax.dot_general(w2t_ref[...], h, (((1,), (0,)), ((), ())),
                        preferred_element_type=jnp.float32)
    h = jnp.maximum(h, 0.0)
    o_ref[...] = lax.dot_general(w3t_ref[...], h, (((1,), (0,)), ((), ())),
                                 preferred_element_type=jnp.float32)


_BN = 32768


def _mlp(enc, w1ta, w1tb, w2t, w3t):
    npts = enc.shape[1]
    bn = min(_BN, npts)
    return pl.pallas_call(
        _mlp_body,
        grid=(npts // bn,),
        in_specs=[
            pl.BlockSpec((_N_LEVELS, bn), lambda i: (0, i)),
            pl.BlockSpec((64, _N_LEVELS), lambda i: (0, 0)),
            pl.BlockSpec((64, _N_LEVELS), lambda i: (0, 0)),
            pl.BlockSpec((64, 64), lambda i: (0, 0)),
            pl.BlockSpec((3, 64), lambda i: (0, 0)),
        ],
        out_specs=pl.BlockSpec((3, bn), lambda i: (0, i)),
        out_shape=jax.ShapeDtypeStruct((3, npts), jnp.float32),
    )(enc, w1ta, w1tb, w2t, w3t)


_NSPLIT = 1


def kernel(uv_coords, bake, table, W1, W2, W3):
    del bake
    n = uv_coords.shape[0]
    ux = uv_coords[:, 0]
    uy = uv_coords[:, 1]
    tabp = lax.bitcast_convert_type(table.astype(jnp.bfloat16), jnp.int32)
    w1t = W1.T
    w1ta, w1tb = w1t[:, 0::2], w1t[:, 1::2]
    w2t, w3t = W2.T, W3.T
    outs = []
    for part in range(_NSPLIT):
        enc = _make_encode(n, _NSPLIT, part)(ux, uy, tabp)
        outs.append(_mlp(enc, w1ta, w1tb, w2t, w3t).T)
    out = jnp.concatenate(outs, axis=0) if len(outs) > 1 else outs[0]
    return out.astype(jnp.float32)
